# Initial kernel scaffold; baseline (speedup 1.0000x reference)
import functools

import jax
import jax.numpy as jnp
from jax import lax
from jax.experimental import pallas as pl
from jax.experimental.pallas import tpu as pltpu
from jax.experimental.pallas import tpu_sc as plsc

NC = 2
NS = 16
NW = NC * NS

B = 16384
L = 50
D = 32
G = 16
ROW = 100
ROWS_PER_CHUNK = (G * L) // ROW
BAGS_PER_WORKER = B // NW
CHUNKS = BAGS_PER_WORKER // G
IDX_ROWS_PER_WORKER = BAGS_PER_WORKER * L // ROW


def _sc_bag_mean(weight, idx_rows, pat_rows):
    mesh = plsc.VectorSubcoreMesh(core_axis_name="c", subcore_axis_name="s")

    @functools.partial(
        pl.kernel,
        mesh=mesh,
        out_type=jax.ShapeDtypeStruct((B, D), jnp.float32),
        scratch_types=[
            pltpu.VMEM((ROWS_PER_CHUNK, ROW), jnp.int32),
            pltpu.VMEM((ROWS_PER_CHUNK, ROW), jnp.int32),
            pltpu.VMEM((G * L, D), jnp.float32),
            pltpu.VMEM((G, D), jnp.float32),
            pltpu.VMEM((G, D), jnp.float32),
            pltpu.VMEM_SHARED((NS * G, D), jnp.float32),
        ],
    )
    def k(table_hbm, idx_hbm, pat_hbm, out_hbm,
          idx_v, bag_v, rows_v, stage_v, zeros_v, acc_sh):
        cid = lax.axis_index("c")
        sid = lax.axis_index("s")
        wid = sid * NC + cid

        pltpu.sync_copy(pat_hbm.at[sid], bag_v)

        @pl.loop(0, G)
        def _(i):
            for h in range(D // 16):
                zeros_v[i, pl.ds(h * 16, 16)] = jnp.zeros((16,), jnp.float32)

        @pl.loop(0, CHUNKS)
        def _(g):
            idx_row_base = wid * IDX_ROWS_PER_WORKER + g * ROWS_PER_CHUNK
            pltpu.sync_copy(idx_hbm.at[pl.ds(idx_row_base, ROWS_PER_CHUNK)],
                            idx_v)
            for j in range(ROWS_PER_CHUNK):
                pltpu.sync_copy(table_hbm.at[idx_v.at[j]],
                                rows_v.at[pl.ds(j * ROW, ROW)])
            pltpu.sync_copy(zeros_v, acc_sh.at[pl.ds(sid * G, G)])
            for j in range(ROWS_PER_CHUNK):
                pltpu.sync_copy(rows_v.at[pl.ds(j * ROW, ROW)],
                                acc_sh.at[bag_v.at[j]], add=True)
            pltpu.sync_copy(acc_sh.at[pl.ds(sid * G, G)], stage_v)

            @pl.loop(0, G)
            def _(i):
                for h in range(D // 16):
                    sl = (i, pl.ds(h * 16, 16))
                    stage_v[sl] = stage_v[sl] * jnp.float32(1.0 / L)

            pltpu.sync_copy(stage_v, out_hbm.at[pl.ds(wid * BAGS_PER_WORKER + g * G, G)])

    return k(weight, idx_rows, pat_rows)


def kernel(input, weight):
    idx_rows = input.astype(jnp.int32).reshape(B * L // ROW, ROW)
    base = jnp.arange(G * L, dtype=jnp.int32) // L
    pat = base[None, :] + (jnp.arange(NS, dtype=jnp.int32) * G)[:, None]
    pat_rows = pat.reshape(NS, ROWS_PER_CHUNK, ROW)
    return _sc_bag_mean(weight, idx_rows, pat_rows)


# speedup vs baseline: 2.1110x; 2.1110x over previous
import functools

import jax
import jax.numpy as jnp
from jax import lax
from jax.experimental import pallas as pl
from jax.experimental.pallas import tpu as pltpu
from jax.experimental.pallas import tpu_sc as plsc

NC = 2
NS = 16
NW = NC * NS

B = 16384
L = 50
D = 32
G = 16
ROW = 100
ROWS_PER_CHUNK = (G * L) // ROW
BAGS_PER_WORKER = B // NW
CHUNKS = BAGS_PER_WORKER // G
IDX_ROWS_PER_WORKER = BAGS_PER_WORKER * L // ROW


def _sc_bag_mean(weight, idx_rows, pat_rows):
    mesh = plsc.VectorSubcoreMesh(core_axis_name="c", subcore_axis_name="s",
                                  num_cores=NC, num_subcores=NS)

    @functools.partial(
        pl.kernel,
        mesh=mesh,
        compiler_params=pltpu.CompilerParams(use_tc_tiling_on_sc=False),
        out_type=jax.ShapeDtypeStruct((B, D), jnp.float32),
        scratch_types=[
            pltpu.VMEM((ROWS_PER_CHUNK, ROW), jnp.int32),
            pltpu.VMEM((ROWS_PER_CHUNK, ROW), jnp.int32),
            pltpu.VMEM((G * L, D), jnp.float32),
            pltpu.VMEM((G, D), jnp.float32),
            pltpu.VMEM((ROW, D), jnp.float32),
            pltpu.VMEM_SHARED((NS * G, D), jnp.float32),
        ],
    )
    def k(table_hbm, idx_hbm, pat_hbm, out_hbm,
          idx_v, bag_v, rows_v, stage_v, zeros_v, acc_sh):
        cid = lax.axis_index("c")
        sid = lax.axis_index("s")
        wid = sid * NC + cid

        pltpu.sync_copy(pat_hbm.at[sid], bag_v)

        @pl.loop(0, ROW)
        def _(i):
            for h in range(D // 16):
                zeros_v[i, pl.ds(h * 16, 16)] = jnp.zeros((16,), jnp.float32)

        @pl.loop(0, CHUNKS)
        def _(g):
            idx_row_base = wid * IDX_ROWS_PER_WORKER + g * ROWS_PER_CHUNK
            pltpu.sync_copy(idx_hbm.at[pl.ds(idx_row_base, ROWS_PER_CHUNK)],
                            idx_v)
            for j in range(ROWS_PER_CHUNK):
                pltpu.sync_copy(table_hbm.at[idx_v.at[j]],
                                rows_v.at[pl.ds(j * ROW, ROW)])
            pltpu.sync_copy(zeros_v.at[pl.ds(0, G)],
                            acc_sh.at[pl.ds(sid * G, G)])
            for j in range(ROWS_PER_CHUNK):
                pltpu.sync_copy(rows_v.at[pl.ds(j * ROW, ROW)],
                                acc_sh.at[bag_v.at[j]], add=True)
            for j in range(2):
                pltpu.sync_copy(zeros_v, acc_sh.at[bag_v.at[j]], add=True)
            pltpu.sync_copy(acc_sh.at[pl.ds(sid * G, G)], stage_v)

            @pl.loop(0, G)
            def _(i):
                for h in range(D // 16):
                    sl = (i, pl.ds(h * 16, 16))
                    stage_v[sl] = stage_v[sl] * jnp.float32(1.0 / L)

            pltpu.sync_copy(stage_v, out_hbm.at[pl.ds(wid * BAGS_PER_WORKER + g * G, G)])

    return k(weight, idx_rows, pat_rows)


def kernel(input, weight):
    idx_rows = input.astype(jnp.int32).reshape(B * L // ROW, ROW)
    base = jnp.arange(G * L, dtype=jnp.int32) // L
    pat = base[None, :] + (jnp.arange(NS, dtype=jnp.int32) * G)[:, None]
    pat_rows = pat.reshape(NS, ROWS_PER_CHUNK, ROW)
    return _sc_bag_mean(weight, idx_rows, pat_rows)


# speedup vs baseline: 2.7570x; 1.3060x over previous
import functools

import jax
import jax.numpy as jnp
from jax import lax
from jax.experimental import pallas as pl
from jax.experimental.pallas import tpu as pltpu
from jax.experimental.pallas import tpu_sc as plsc

NC = 2
NS = 16
NW = NC * NS

B = 16384
L = 50
D = 32
G = 32
ROW = 100
RPC = (G * L) // ROW
BAGS_PER_WORKER = B // NW
CHUNKS = BAGS_PER_WORKER // G
IDX_ROWS_PER_WORKER = BAGS_PER_WORKER * L // ROW
NPAD = 4


def _sc_bag_mean(weight, idx_rows, pat_rows):
    mesh = plsc.VectorSubcoreMesh(core_axis_name="c", subcore_axis_name="s",
                                  num_cores=NC, num_subcores=NS)

    @functools.partial(
        pl.kernel,
        mesh=mesh,
        compiler_params=pltpu.CompilerParams(use_tc_tiling_on_sc=False),
        out_type=jax.ShapeDtypeStruct((B, D), jnp.float32),
        scratch_types=[
            pltpu.VMEM((RPC, ROW), jnp.int32),
            pltpu.VMEM((RPC, ROW), jnp.int32),
            pltpu.VMEM((2, RPC, ROW), jnp.int32),
            pltpu.VMEM((G * L, D), jnp.float32),
            pltpu.VMEM((G * L, D), jnp.float32),
            pltpu.VMEM((G, D), jnp.float32),
            pltpu.VMEM((G, D), jnp.float32),
            pltpu.VMEM((ROW, D), jnp.float32),
            pltpu.VMEM_SHARED((NS * 2 * G, D), jnp.float32),
        ] + [pltpu.SemaphoreType.DMA] * 10,
    )
    def k(table_hbm, idx_hbm, pat_hbm, out_hbm,
          idx0, idx1, bag_v, rows0, rows1, st0, st1, zeros_v, acc_sh,
          s_idx0, s_idx1, s_g0, s_g1, s_add0, s_add1, s_z0, s_z1,
          s_out0, s_out1):
        cid = lax.axis_index("c")
        sid = lax.axis_index("s")
        wid = sid * NC + cid

        idx_b = [idx0, idx1]
        rows_b = [rows0, rows1]
        stage_b = [st0, st1]
        s_idx = [s_idx0, s_idx1]
        s_g = [s_g0, s_g1]
        s_add = [s_add0, s_add1]
        s_z = [s_z0, s_z1]
        s_out = [s_out0, s_out1]

        def region(parity):
            return pl.ds((sid * 2 + parity) * G, G)

        pltpu.sync_copy(pat_hbm.at[sid], bag_v)

        @pl.loop(0, ROW)
        def _(i):
            for h in range(D // 16):
                zeros_v[i, pl.ds(h * 16, 16)] = jnp.zeros((16,), jnp.float32)

        for p in range(2):
            pltpu.sync_copy(zeros_v.at[pl.ds(0, G)], acc_sh.at[region(p)])

        hs = {}
        for p in range(2):
            base = wid * IDX_ROWS_PER_WORKER + p * RPC
            hs["idx", p] = pltpu.async_copy(
                idx_hbm.at[pl.ds(base, RPC)], idx_b[p], s_idx[p])

        def add_phase(m, npad):
            q = m % 2
            for h in hs["g", q]:
                h.wait()
            if m + 2 < CHUNKS:
                base = wid * IDX_ROWS_PER_WORKER + (m + 2) * RPC
                hs["idx", q] = pltpu.async_copy(
                    idx_hbm.at[pl.ds(base, RPC)], idx_b[q], s_idx[q])
            if ("z", q) in hs:
                hs.pop(("z", q)).wait()
            adds = []
            for j in range(RPC):
                adds.append(pltpu.async_copy(
                    rows_b[q].at[pl.ds(j * ROW, ROW)],
                    acc_sh.at[bag_v.at[q].at[j]], s_add[q], add=True))
            for j in range(npad):
                adds.append(pltpu.async_copy(
                    zeros_v, acc_sh.at[bag_v.at[q].at[0]], s_add[q],
                    add=True))
            for h in adds:
                h.wait()

        def read_phase(m):
            q = m % 2
            if ("out", q) in hs:
                hs.pop(("out", q)).wait()
            pltpu.sync_copy(acc_sh.at[region(q)], stage_b[q])
            hs["z", q] = pltpu.async_copy(
                zeros_v.at[pl.ds(0, G)], acc_sh.at[region(q)], s_z[q])

            @pl.loop(0, G)
            def _(i):
                for h in range(D // 16):
                    sl = (i, pl.ds(h * 16, 16))
                    stage_b[q][sl] = stage_b[q][sl] * jnp.float32(1.0 / L)

            hs["out", q] = pltpu.async_copy(
                stage_b[q],
                out_hbm.at[pl.ds(wid * BAGS_PER_WORKER + m * G, G)],
                s_out[q])

        for g in range(CHUNKS):
            p = g % 2
            hs["idx", p].wait()
            hs["g", p] = [
                pltpu.async_copy(table_hbm.at[idx_b[p].at[j]],
                                 rows_b[p].at[pl.ds(j * ROW, ROW)], s_g[p])
                for j in range(RPC)]
            if g >= 1:
                add_phase(g - 1, npad=0)
            if g >= 2:
                read_phase(g - 2)
        add_phase(CHUNKS - 1, npad=NPAD)
        read_phase(CHUNKS - 2)
        read_phase(CHUNKS - 1)

        for q in range(2):
            hs["z", q].wait()
            hs["out", q].wait()

    return k(weight, idx_rows, pat_rows)


def kernel(input, weight):
    idx_rows = input.astype(jnp.int32).reshape(B * L // ROW, ROW)
    local_bag = jnp.arange(G * L, dtype=jnp.int32) // L
    sidb = (jnp.arange(NS, dtype=jnp.int32) * 2)[:, None, None]
    par = jnp.arange(2, dtype=jnp.int32)[None, :, None]
    pat = (sidb + par) * G + local_bag[None, None, :]
    pat_rows = pat.reshape(NS, 2, RPC, ROW)
    return _sc_bag_mean(weight, idx_rows, pat_rows)


# speedup vs baseline: 2.7657x; 1.0032x over previous
import functools

import jax
import jax.numpy as jnp
from jax import lax
from jax.experimental import pallas as pl
from jax.experimental.pallas import tpu as pltpu
from jax.experimental.pallas import tpu_sc as plsc

NC = 2
NS = 16
NW = NC * NS

B = 16384
L = 50
D = 32
NUM_ROWS = 1000000
G = 32
ROW = 100
RPC = (G * L) // ROW
GD = 80
NGD = (G * L) // GD
IPC = G * L
BAGS_PER_WORKER = B // NW
CHUNKS = BAGS_PER_WORKER // G
NPAD = 4


def _sc_bag_mean(weight, idx_rows, pat_rows):
    mesh = plsc.VectorSubcoreMesh(core_axis_name="c", subcore_axis_name="s",
                                  num_cores=NC, num_subcores=NS)

    @functools.partial(
        pl.kernel,
        mesh=mesh,
        compiler_params=pltpu.CompilerParams(use_tc_tiling_on_sc=False),
        out_type=jax.ShapeDtypeStruct((B, D), jnp.float32),
        scratch_types=[
            pltpu.VMEM((IPC,), jnp.int32),
            pltpu.VMEM((IPC,), jnp.int32),
            pltpu.VMEM((2, RPC, ROW), jnp.int32),
            pltpu.VMEM((G * L, D), jnp.float32),
            pltpu.VMEM((G * L, D), jnp.float32),
            pltpu.VMEM((G, D), jnp.float32),
            pltpu.VMEM((G, D), jnp.float32),
            pltpu.VMEM((ROW, D), jnp.float32),
            pltpu.VMEM_SHARED((NS * 2 * G, D), jnp.float32),
        ] + [pltpu.SemaphoreType.DMA] * 10,
    )
    def k(table1d_hbm, idx_hbm, pat_hbm, out_hbm,
          idx0, idx1, bag_v, rows0, rows1, st0, st1, zeros_v, acc_sh,
          s_idx0, s_idx1, s_g0, s_g1, s_add0, s_add1, s_z0, s_z1,
          s_out0, s_out1):
        table_hbm = table1d_hbm
        cid = lax.axis_index("c")
        sid = lax.axis_index("s")
        wid = sid * NC + cid

        idx_b = [idx0, idx1]
        rows_b = [rows0, rows1]
        stage_b = [st0, st1]
        s_idx = [s_idx0, s_idx1]
        s_g = [s_g0, s_g1]
        s_add = [s_add0, s_add1]
        s_z = [s_z0, s_z1]
        s_out = [s_out0, s_out1]

        def region(parity):
            return pl.ds((sid * 2 + parity) * G, G)

        pltpu.sync_copy(pat_hbm.at[sid], bag_v)

        @pl.loop(0, ROW)
        def _(i):
            for h in range(D // 16):
                zeros_v[i, pl.ds(h * 16, 16)] = jnp.zeros((16,), jnp.float32)

        for p in range(2):
            pltpu.sync_copy(zeros_v.at[pl.ds(0, G)], acc_sh.at[region(p)])

        hs = {}
        for p in range(2):
            base = wid * (BAGS_PER_WORKER * L) + p * IPC
            hs["idx", p] = pltpu.async_copy(
                idx_hbm.at[pl.ds(base, IPC)], idx_b[p], s_idx[p])

        def add_phase(m, npad):
            q = m % 2
            for h in hs["g", q]:
                h.wait()
            if m + 2 < CHUNKS:
                base = wid * (BAGS_PER_WORKER * L) + (m + 2) * IPC
                hs["idx", q] = pltpu.async_copy(
                    idx_hbm.at[pl.ds(base, IPC)], idx_b[q], s_idx[q])
            if ("z", q) in hs:
                hs.pop(("z", q)).wait()
            adds = []
            for j in range(RPC):
                adds.append(pltpu.async_copy(
                    rows_b[q].at[pl.ds(j * ROW, ROW)],
                    acc_sh.at[bag_v.at[q].at[j]], s_add[q], add=True))
            for j in range(npad):
                adds.append(pltpu.async_copy(
                    zeros_v, acc_sh.at[bag_v.at[q].at[0]], s_add[q],
                    add=True))
            for h in adds:
                h.wait()

        def read_phase(m):
            q = m % 2
            if ("out", q) in hs:
                hs.pop(("out", q)).wait()
            pltpu.sync_copy(acc_sh.at[region(q)], stage_b[q])
            hs["z", q] = pltpu.async_copy(
                zeros_v.at[pl.ds(0, G)], acc_sh.at[region(q)], s_z[q])

            @pl.loop(0, G)
            def _(i):
                for h in range(D // 16):
                    sl = (i, pl.ds(h * 16, 16))
                    stage_b[q][sl] = stage_b[q][sl] * jnp.float32(1.0 / L)

            hs["out", q] = pltpu.async_copy(
                stage_b[q],
                out_hbm.at[pl.ds(wid * BAGS_PER_WORKER + m * G, G)],
                s_out[q])

        for g in range(CHUNKS):
            p = g % 2
            hs["idx", p].wait()
            hs["g", p] = [
                pltpu.async_copy(table_hbm.at[idx_b[p].at[pl.ds(j * GD, GD)]],
                                 rows_b[p].at[pl.ds(j * GD, GD)], s_g[p])
                for j in range(NGD)]
            if g >= 1:
                add_phase(g - 1, npad=0)
            if g >= 2:
                read_phase(g - 2)
        add_phase(CHUNKS - 1, npad=NPAD)
        read_phase(CHUNKS - 2)
        read_phase(CHUNKS - 1)

        for q in range(2):
            hs["z", q].wait()
            hs["out", q].wait()

    return k(weight, idx_rows, pat_rows)


def kernel(input, weight):
    idx_rows = input.astype(jnp.int32).reshape(-1)
    local_bag = jnp.arange(G * L, dtype=jnp.int32) // L
    sidb = (jnp.arange(NS, dtype=jnp.int32) * 2)[:, None, None]
    par = jnp.arange(2, dtype=jnp.int32)[None, :, None]
    pat = (sidb + par) * G + local_bag[None, None, :]
    pat_rows = pat.reshape(NS, 2, RPC, ROW)
    return _sc_bag_mean(weight, idx_rows, pat_rows)
